# Initial kernel scaffold; baseline (speedup 1.0000x reference)
import jax
import jax.numpy as jnp
from jax.experimental import pallas as pl


def kernel(ratings, w):
    raise NotImplementedError("write your pallas kernel here")


# speedup vs baseline: 12.3983x; 12.3983x over previous
import jax
import jax.numpy as jnp
from jax.experimental import pallas as pl
from jax.experimental.pallas import tpu as pltpu

_K = 10
_FILL = float(jnp.finfo(jnp.float32).min)
_NEG = float("-inf")
_IDBITS = 17


def _rec_kernel(w_ref, r_ref, out_ref):
    RB, N = out_ref.shape
    x0 = r_ref[0]
    x1 = r_ref[1]
    iota = jax.lax.broadcasted_iota(jnp.int32, (RB, N), 1)

    def topk_ids(x):
        work = x
        ids = []
        for _ in range(_K):
            mx = jnp.max(work, axis=1, keepdims=True)
            am = jnp.min(jnp.where(work == mx, iota, N), axis=1, keepdims=True)
            ids.append(am)
            work = jnp.where(iota == am, _NEG, work)
        return jnp.concatenate(ids, axis=1)

    ids0 = topk_ids(x0)
    ids1 = topk_ids(x1)

    eq = ids0[:, :, None] == ids1[:, None, :]
    dup0 = jnp.sum(eq.astype(jnp.int32), axis=2)
    dup1 = jnp.sum(eq.astype(jnp.int32), axis=1)
    cand_ids = jnp.concatenate([ids0, ids1], axis=1)
    cnt = jnp.concatenate([1 + dup0, 1 + dup1], axis=1)
    valid = jnp.concatenate([jnp.ones_like(dup1), 1 - dup1], axis=1)
    key = jnp.where(valid > 0, (cnt << _IDBITS) - cand_ids, -(1 << 30))
    jidx = jax.lax.broadcasted_iota(jnp.int32, key.shape, 1)

    w0 = w_ref[0, 0]
    w1 = w_ref[0, 1]
    xc = w0 * x0 + w1 * x1

    selmask = jnp.zeros((RB, N), dtype=jnp.bool_)
    for _ in range(_K):
        mk = jnp.max(key, axis=1, keepdims=True)
        amj = jnp.min(jnp.where(key == mk, jidx, 2 * _K), axis=1, keepdims=True)
        sel = jnp.sum(jnp.where(jidx == amj, cand_ids, 0), axis=1, keepdims=True)
        selmask = jnp.logical_or(selmask, iota == sel)
        key = jnp.where(jidx == amj, -(1 << 30), key)

    out_ref[...] = jnp.where(selmask, xc, _FILL)


def kernel(ratings, w):
    M, B, N = ratings.shape
    RB = 8
    w2 = w.reshape(1, M).astype(jnp.float32)
    return pl.pallas_call(
        _rec_kernel,
        grid=(B // RB,),
        in_specs=[
            pl.BlockSpec(memory_space=pltpu.SMEM),
            pl.BlockSpec((M, RB, N), lambda i: (0, i, 0)),
        ],
        out_specs=pl.BlockSpec((RB, N), lambda i: (i, 0)),
        out_shape=jax.ShapeDtypeStruct((B, N), jnp.float32),
    )(w2, ratings)
